# baseline (speedup 1.0000x reference)
import functools

import jax
import jax.numpy as jnp
from jax import lax
from jax.experimental import pallas as pl
from jax.experimental.pallas import tpu as pltpu
from jax.experimental.pallas import tpu_sc as plsc

NC = 2
NS = 16
NW = NC * NS

CHUNK = 2048
NBUF = 2
GROUP = NBUF * CHUNK


def _gather_body(x_hbm, table_hbm, out_hbm, idx_v, rows_v, *sems,
                 per_w, n_groups):
  sem_i = sems[0:2]
  sem_g = sems[2:2 + NBUF]
  sem_s = sems[2 + NBUF:]
  wid = lax.axis_index("s") * NC + lax.axis_index("c")
  base_w = wid * per_w

  def start_idx(g, p):
    pltpu.async_copy(x_hbm.at[pl.ds(base_w + g * GROUP, GROUP)], idx_v.at[p],
                     sem_i[p])

  def wait_idx(p):
    pltpu.make_async_copy(x_hbm.at[pl.ds(0, GROUP)], idx_v.at[p],
                          sem_i[p]).wait()

  def start_gather(p, b):
    pltpu.async_copy(table_hbm.at[idx_v.at[p, pl.ds(b * CHUNK, CHUNK)]],
                     rows_v.at[b], sem_g[b])

  def wait_gather(p, b):
    pltpu.make_async_copy(table_hbm.at[idx_v.at[p, pl.ds(b * CHUNK, CHUNK)]],
                          rows_v.at[b], sem_g[b]).wait()

  def start_store(g, b):
    pltpu.async_copy(
        rows_v.at[b],
        out_hbm.at[pl.ds(base_w + g * GROUP + b * CHUNK, CHUNK)], sem_s[b])

  def wait_store(b):
    pltpu.make_async_copy(rows_v.at[b], out_hbm.at[pl.ds(0, CHUNK)],
                          sem_s[b]).wait()

  start_idx(0, 0)
  wait_idx(0)
  for b in range(NBUF):
    start_gather(0, b)
  if n_groups > 1:
    start_idx(1, 1)

  def do_group(g, p, prefetch):
    q = 1 - p
    for b in range(NBUF):
      wait_gather(p, b)
      start_store(g, b)
    if prefetch:
      start_idx(g + 2, p)
    wait_idx(q)
    for b in range(NBUF):
      wait_store(b)
      start_gather(q, b)

  n_pre = n_groups - 2
  pairs = n_pre // 2

  def body(i, carry):
    do_group(2 * i, 0, True)
    do_group(2 * i + 1, 1, True)
    return carry

  if pairs > 0:
    lax.fori_loop(0, pairs, body, 0)
  g0 = 2 * pairs
  if n_pre - g0 == 1:
    do_group(g0, g0 % 2, True)
    g0 += 1

  do_group(g0, g0 % 2, False)

  g = n_groups - 1
  p = g % 2
  for b in range(NBUF):
    wait_gather(p, b)
    start_store(g, b)
  for b in range(NBUF):
    wait_store(b)


def kernel(x, table):
  batch, num_node = x.shape
  dim = table.shape[1]
  flat = x.reshape(-1).astype(jnp.int32)
  b = flat.shape[0]
  assert b % NW == 0
  per_w = b // NW
  assert per_w % GROUP == 0
  n_groups = per_w // GROUP

  mesh = plsc.VectorSubcoreMesh(
      core_axis_name="c", subcore_axis_name="s", num_cores=NC, num_subcores=NS
  )
  out = pl.kernel(
      functools.partial(_gather_body, per_w=per_w, n_groups=n_groups),
      out_type=jax.ShapeDtypeStruct((b, dim), jnp.float32),
      mesh=mesh,
      scratch_types=(
          [pltpu.VMEM((2, GROUP), jnp.int32),
           pltpu.VMEM((NBUF, CHUNK, dim), jnp.float32)]
          + [pltpu.SemaphoreType.DMA] * (2 + 2 * NBUF)
      ),
      compiler_params=pltpu.CompilerParams(use_tc_tiling_on_sc=False),
  )(flat, table)
  return out.reshape(batch, num_node, dim)


# speedup vs baseline: 1.0037x; 1.0037x over previous
import functools

import jax
import jax.numpy as jnp
from jax import lax
from jax.experimental import pallas as pl
from jax.experimental.pallas import tpu as pltpu
from jax.experimental.pallas import tpu_sc as plsc

NC = 2
NS = 16
NW = NC * NS

CHUNK = 512
NBUF = 8
GROUP = NBUF * CHUNK


def _gather_body(x_hbm, table_hbm, out_hbm, idx_v, rows_v, *sems,
                 per_w, n_groups):
  sem_i = sems[0:2]
  sem_g = sems[2:2 + NBUF]
  sem_s = sems[2 + NBUF:]
  wid = lax.axis_index("s") * NC + lax.axis_index("c")
  base_w = wid * per_w

  def start_idx(g, p):
    pltpu.async_copy(x_hbm.at[pl.ds(base_w + g * GROUP, GROUP)], idx_v.at[p],
                     sem_i[p])

  def wait_idx(p):
    pltpu.make_async_copy(x_hbm.at[pl.ds(0, GROUP)], idx_v.at[p],
                          sem_i[p]).wait()

  def start_gather(p, b):
    pltpu.async_copy(table_hbm.at[idx_v.at[p, pl.ds(b * CHUNK, CHUNK)]],
                     rows_v.at[b], sem_g[b])

  def wait_gather(p, b):
    pltpu.make_async_copy(table_hbm.at[idx_v.at[p, pl.ds(b * CHUNK, CHUNK)]],
                          rows_v.at[b], sem_g[b]).wait()

  def start_store(g, b):
    pltpu.async_copy(
        rows_v.at[b],
        out_hbm.at[pl.ds(base_w + g * GROUP + b * CHUNK, CHUNK)], sem_s[b])

  def wait_store(b):
    pltpu.make_async_copy(rows_v.at[b], out_hbm.at[pl.ds(0, CHUNK)],
                          sem_s[b]).wait()

  start_idx(0, 0)
  wait_idx(0)
  for b in range(NBUF):
    start_gather(0, b)
  if n_groups > 1:
    start_idx(1, 1)

  def do_group(g, p, prefetch):
    q = 1 - p
    for b in range(NBUF):
      wait_gather(p, b)
      start_store(g, b)
    if prefetch:
      start_idx(g + 2, p)
    wait_idx(q)
    for b in range(NBUF):
      wait_store(b)
      start_gather(q, b)

  n_pre = n_groups - 2
  pairs = n_pre // 2

  def body(i, carry):
    do_group(2 * i, 0, True)
    do_group(2 * i + 1, 1, True)
    return carry

  if pairs > 0:
    lax.fori_loop(0, pairs, body, 0)
  g0 = 2 * pairs
  if n_pre - g0 == 1:
    do_group(g0, g0 % 2, True)
    g0 += 1

  do_group(g0, g0 % 2, False)

  g = n_groups - 1
  p = g % 2
  for b in range(NBUF):
    wait_gather(p, b)
    start_store(g, b)
  for b in range(NBUF):
    wait_store(b)


def kernel(x, table):
  batch, num_node = x.shape
  dim = table.shape[1]
  flat = x.reshape(-1).astype(jnp.int32)
  b = flat.shape[0]
  assert b % NW == 0
  per_w = b // NW
  assert per_w % GROUP == 0
  n_groups = per_w // GROUP

  mesh = plsc.VectorSubcoreMesh(
      core_axis_name="c", subcore_axis_name="s", num_cores=NC, num_subcores=NS
  )
  out = pl.kernel(
      functools.partial(_gather_body, per_w=per_w, n_groups=n_groups),
      out_type=jax.ShapeDtypeStruct((b, dim), jnp.float32),
      mesh=mesh,
      scratch_types=(
          [pltpu.VMEM((2, GROUP), jnp.int32),
           pltpu.VMEM((NBUF, CHUNK, dim), jnp.float32)]
          + [pltpu.SemaphoreType.DMA] * (2 + 2 * NBUF)
      ),
      compiler_params=pltpu.CompilerParams(use_tc_tiling_on_sc=False),
  )(flat, table)
  return out.reshape(batch, num_node, dim)
